# baseline (speedup 1.0000x reference)
import jax
import jax.numpy as jnp
from jax import lax
from jax.experimental import pallas as pl
from jax.experimental.pallas import tpu as pltpu
from jax.experimental.pallas import tpu_sc as plsc

B = 4
N = 20000
M = 20
NPAD = 20480
NROW = NPAD // 128
L = 16
BIG_IDX = 1 << 30


def _tc_scan_body(comp_ref, tgt_ref, mx_ref, idx_ref):
    f32 = jnp.float32
    px = comp_ref[0, 0]
    py = comp_ref[1, 0]
    pw = comp_ref[2, 0]
    ph = comp_ref[3, 0]
    cf = comp_ref[4, 0]
    px2 = px + pw
    py2 = py + ph
    pA = pw * ph

    row = lax.broadcasted_iota(jnp.int32, (NROW, 128), 0)
    col = lax.broadcasted_iota(jnp.int32, (NROW, 128), 1)
    lin = row * 128 + col
    sub = lax.broadcasted_iota(jnp.int32, (8, 128), 0)
    ln = lax.broadcasted_iota(jnp.int32, (8, 128), 1)

    macc = jnp.zeros((8, 128), f32)
    iacc = jnp.zeros((8, 128), jnp.int32)
    for m in range(M):
        gx = tgt_ref[0, 0, m]
        gy = tgt_ref[0, 1, m]
        gw = tgt_ref[0, 2, m]
        gh = tgt_ref[0, 3, m]
        xa = jnp.maximum(px, gx)
        ya = jnp.maximum(py, gy)
        xb = jnp.minimum(px2, gx + gw)
        yb = jnp.minimum(py2, gy + gh)
        inter = jnp.maximum(xb - xa, 0.0) * jnp.maximum(yb - ya, 0.0)
        union = (pA + gw * gh) - inter
        upos = union > 0.0
        iou = jnp.where(upos, inter / jnp.where(upos, union, 1.0), 0.0)
        mx = jnp.max(iou)
        mn = jnp.min(jnp.where(iou == mx, lin, BIG_IDX))
        maskm = (sub == 0) & (ln == m)
        macc = jnp.where(maskm, mx, macc)
        iacc = jnp.where(maskm, mn, iacc)

    s0 = jnp.sum(jnp.maximum(cf, 0.0) + jnp.log1p(jnp.exp(-jnp.abs(cf))))
    macc = jnp.where((sub == 1) & (ln == 0), s0, macc)

    mx_ref[0] = macc
    idx_ref[0] = iacc


def _sc_select_body(comp_hbm, tgt_hbm, mx_hbm, idx_hbm, out_hbm,
                    px, py, pw, ph, cf, tg, mrow, irow, outrow):
    c = lax.axis_index("c")
    s = lax.axis_index("s")

    @pl.when((s == 0) | (s == 8))
    def _select():
        b = c * 2 + s // 8
        for k, ref in ((0, px), (1, py), (2, pw), (3, ph), (4, cf)):
            pltpu.sync_copy(comp_hbm.at[pl.ds((k * B + b) * NPAD, NPAD)], ref)
        pltpu.sync_copy(tgt_hbm.at[pl.ds(b * 128, 128)], tg)
        pltpu.sync_copy(mx_hbm.at[pl.ds(b * 1024, 32)], mrow)
        pltpu.sync_copy(idx_hbm.at[pl.ds(b * 1024, 32)], irow)

        lane = lax.broadcasted_iota(jnp.int32, (L,), 0)
        gmax = [mrow[pl.ds(0, L)], mrow[pl.ds(L, L)]]
        gidx = [irow[pl.ds(0, L)], irow[pl.ds(L, L)]]
        gbox = [[plsc.load_gather(ref, [gidx[h]]) for h in range(2)]
                for ref in (px, py, pw, ph, cf)]

        hit = [gmax[h] > 0.5 for h in range(2)]
        hiti = [hit[h].astype(jnp.int32) for h in range(2)]

        mpos = [lane, lane + L]
        dup = [jnp.zeros((L,), jnp.bool_) for _ in range(2)]
        for mp in range(M):
            jm = gidx[mp // L][mp % L]
            hm = hiti[mp // L][mp % L] > 0
            for h in range(2):
                clash = hm & (gidx[h] == jm) & (mpos[h] > mp)
                dup[h] = dup[h] | clash
        valid = [hit[h] & (~dup[h]) for h in range(2)]
        key = [jnp.where(valid[h], gidx[h], BIG_IDX) for h in range(2)]

        rank = [jnp.zeros((L,), jnp.int32) for _ in range(2)]
        for mp in range(M):
            km = key[mp // L][mp % L]
            for h in range(2):
                rank[h] = rank[h] + (key[h] > km).astype(jnp.int32)

        nval = jnp.sum(valid[0].astype(jnp.int32)) + \
            jnp.sum(valid[1].astype(jnp.int32))
        sx = jnp.sum(jnp.where(valid[0], gbox[4][0], 0.0)) + \
            jnp.sum(jnp.where(valid[1], gbox[4][1], 0.0))

        bbox = jnp.zeros((L,), jnp.float32)
        for h in range(2):
            acc = jnp.zeros((L,), jnp.float32)
            for fi in range(4):
                tcomp = plsc.load_gather(tg, [fi * 32 + rank[h]])
                d = gbox[fi][h] - tcomp
                acc = acc + d * d
            bbox = bbox + jnp.where(valid[h], acc, 0.0)
        bb = jnp.sum(bbox)

        out_v = jnp.where(lane == 0, nval.astype(jnp.float32),
                          jnp.where(lane == 1, sx,
                                    jnp.where(lane == 2, bb, 0.0)))
        outrow[pl.ds(0, L)] = out_v
        pltpu.sync_copy(outrow, out_hbm.at[pl.ds(b * L, L)])


@jax.jit
def kernel(preds, targets):
    f32 = jnp.float32
    comp = jnp.transpose(preds, (2, 0, 1))
    padc = jnp.concatenate([
        jnp.full((2, B, NPAD - N), 2.0, f32),
        jnp.zeros((2, B, NPAD - N), f32),
        jnp.full((1, B, NPAD - N), -1e30, f32),
    ], axis=0)
    comp = jnp.concatenate([comp, padc], axis=2)
    comp_flat = comp.reshape(5 * B * NPAD)
    comp4 = comp.reshape(5, B, NROW, 128)
    tgt = jnp.transpose(targets, (0, 2, 1))
    tgtp = jnp.pad(tgt, ((0, 0), (0, 4), (0, 32 - M)))
    tgt_flat = tgtp[:, :4, :].reshape(B * 4 * 32)
    tgt_tc = jnp.pad(tgtp, ((0, 0), (0, 0), (0, 96)))

    mx_out, idx_out = pl.pallas_call(
        _tc_scan_body,
        out_shape=(jax.ShapeDtypeStruct((B, 8, 128), f32),
                   jax.ShapeDtypeStruct((B, 8, 128), jnp.int32)),
        grid=(B,),
        in_specs=[pl.BlockSpec((5, 1, NROW, 128), lambda i: (0, i, 0, 0)),
                  pl.BlockSpec((1, 8, 128), lambda i: (i, 0, 0))],
        out_specs=(pl.BlockSpec((1, 8, 128), lambda i: (i, 0, 0)),
                   pl.BlockSpec((1, 8, 128), lambda i: (i, 0, 0))),
    )(comp4, tgt_tc)

    mesh = plsc.VectorSubcoreMesh(core_axis_name="c", subcore_axis_name="s")
    sc_call = pl.kernel(
        _sc_select_body,
        out_type=jax.ShapeDtypeStruct((B * L,), f32),
        mesh=mesh,
        compiler_params=pltpu.CompilerParams(needs_layout_passes=False),
        scratch_types=[
            pltpu.VMEM((NPAD,), f32),
            pltpu.VMEM((NPAD,), f32),
            pltpu.VMEM((NPAD,), f32),
            pltpu.VMEM((NPAD,), f32),
            pltpu.VMEM((NPAD,), f32),
            pltpu.VMEM((128,), f32),
            pltpu.VMEM((32,), f32),
            pltpu.VMEM((32,), jnp.int32),
            pltpu.VMEM((L,), f32),
        ],
    )
    sc_out = sc_call(comp_flat, tgt_flat,
                     mx_out.reshape(B * 1024),
                     idx_out.reshape(B * 1024)).reshape(B, L)

    s0 = mx_out[:, 1, 0]
    n = sc_out[:, 0]
    sx = sc_out[:, 1]
    bb = sc_out[:, 2]
    conf_loss = (s0 - sx) / N
    bbox_loss = bb / (jnp.maximum(n, 1.0) * 4.0)
    per_batch = jnp.where(n > 0, bbox_loss + conf_loss, 0.0)
    return jnp.asarray(jnp.mean(per_batch), f32)


# speedup vs baseline: 1.2110x; 1.2110x over previous
import jax
import jax.numpy as jnp
from jax import lax
from jax.experimental import pallas as pl
from jax.experimental.pallas import tpu as pltpu
from jax.experimental.pallas import tpu_sc as plsc

B = 4
N = 20000
M = 20
MH = 10
NPAD = 20480
NROW = NPAD // 128
NK = NROW // 8
L = 16
BIG_IDX = 1 << 30


def _tc_scan_body(comp_ref, tgt_ref, mx_ref, idx_ref):
    f32 = jnp.float32
    sub = lax.broadcasted_iota(jnp.int32, (8, 128), 0)
    ln = lax.broadcasted_iota(jnp.int32, (8, 128), 1)
    sublin = sub * 128 + ln

    rmax = [jnp.full((8, 128), -1.0, f32) for _ in range(MH)]
    ridx = [jnp.zeros((8, 128), jnp.int32) for _ in range(MH)]
    for k in range(NK):
        px = comp_ref[0, 0, pl.ds(k * 8, 8), :]
        py = comp_ref[1, 0, pl.ds(k * 8, 8), :]
        pw = comp_ref[2, 0, pl.ds(k * 8, 8), :]
        ph = comp_ref[3, 0, pl.ds(k * 8, 8), :]
        px2 = px + pw
        py2 = py + ph
        pA = pw * ph
        kidx = k * 1024 + sublin
        for mi in range(MH):
            gx = tgt_ref[0, 0, mi]
            gy = tgt_ref[0, 1, mi]
            gw = tgt_ref[0, 2, mi]
            gh = tgt_ref[0, 3, mi]
            xa = jnp.maximum(px, gx)
            ya = jnp.maximum(py, gy)
            xb = jnp.minimum(px2, gx + gw)
            yb = jnp.minimum(py2, gy + gh)
            inter = jnp.maximum(xb - xa, 0.0) * jnp.maximum(yb - ya, 0.0)
            union = (pA + gw * gh) - inter
            upos = union > 0.0
            iou = jnp.where(upos, inter / jnp.where(upos, union, 1.0), 0.0)
            upd = iou > rmax[mi]
            rmax[mi] = jnp.where(upd, iou, rmax[mi])
            ridx[mi] = jnp.where(upd, kidx, ridx[mi])

    macc = jnp.zeros((8, 128), f32)
    iacc = jnp.zeros((8, 128), jnp.int32)
    for mi in range(MH):
        mx = jnp.max(rmax[mi])
        mn = jnp.min(jnp.where(rmax[mi] == mx, ridx[mi], BIG_IDX))
        maskm = (sub == 0) & (ln == mi)
        macc = jnp.where(maskm, mx, macc)
        iacc = jnp.where(maskm, mn, iacc)

    spacc = jnp.zeros((8, 128), f32)
    for k in range(NK):
        cf = comp_ref[4, 0, pl.ds(k * 8, 8), :]
        spacc = spacc + jnp.maximum(cf, 0.0) + \
            jnp.log1p(jnp.exp(-jnp.abs(cf)))
    macc = jnp.where((sub == 1) & (ln == 0), jnp.sum(spacc), macc)

    mx_ref[0] = macc
    idx_ref[0] = iacc


def _sc_select_body(comp_hbm, tgt_hbm, mx_hbm, idx_hbm, out_hbm,
                    px, py, pw, ph, cf, tg, mrows, irows, outrow):
    c = lax.axis_index("c")
    s = lax.axis_index("s")

    @pl.when((s == 0) | (s == 8))
    def _select():
        b = c * 2 + s // 8
        for k, ref in ((0, px), (1, py), (2, pw), (3, ph), (4, cf)):
            pltpu.sync_copy(comp_hbm.at[pl.ds((k * B + b) * NPAD, NPAD)], ref)
        pltpu.sync_copy(tgt_hbm.at[pl.ds(b * 128, 128)], tg)
        pltpu.sync_copy(mx_hbm.at[pl.ds((2 * b) * 1024, 32)],
                        mrows.at[pl.ds(0, 32)])
        pltpu.sync_copy(mx_hbm.at[pl.ds((2 * b + 1) * 1024, 32)],
                        mrows.at[pl.ds(32, 32)])
        pltpu.sync_copy(idx_hbm.at[pl.ds((2 * b) * 1024, 32)],
                        irows.at[pl.ds(0, 32)])
        pltpu.sync_copy(idx_hbm.at[pl.ds((2 * b + 1) * 1024, 32)],
                        irows.at[pl.ds(32, 32)])

        lane = lax.broadcasted_iota(jnp.int32, (L,), 0)
        sel0 = jnp.where(lane < MH, lane, lane + 22)
        sel1 = jnp.minimum(lane + 38, 63)
        in1 = lane < M - 16
        gmax = [plsc.load_gather(mrows, [sel0]),
                jnp.where(in1, plsc.load_gather(mrows, [sel1]), -1.0)]
        gidx = [plsc.load_gather(irows, [sel0]),
                jnp.where(in1, plsc.load_gather(irows, [sel1]), 0)]
        gbox = [[plsc.load_gather(ref, [gidx[h]]) for h in range(2)]
                for ref in (px, py, pw, ph, cf)]

        hit = [gmax[h] > 0.5 for h in range(2)]
        hiti = [hit[h].astype(jnp.int32) for h in range(2)]

        mpos = [lane, lane + L]
        dup = [jnp.zeros((L,), jnp.bool_) for _ in range(2)]
        for mp in range(M):
            jm = gidx[mp // L][mp % L]
            hm = hiti[mp // L][mp % L] > 0
            for h in range(2):
                clash = hm & (gidx[h] == jm) & (mpos[h] > mp)
                dup[h] = dup[h] | clash
        valid = [hit[h] & (~dup[h]) for h in range(2)]
        key = [jnp.where(valid[h], gidx[h], BIG_IDX) for h in range(2)]

        rank = [jnp.zeros((L,), jnp.int32) for _ in range(2)]
        for mp in range(M):
            km = key[mp // L][mp % L]
            for h in range(2):
                rank[h] = rank[h] + (key[h] > km).astype(jnp.int32)

        nval = jnp.sum(valid[0].astype(jnp.int32)) + \
            jnp.sum(valid[1].astype(jnp.int32))
        sx = jnp.sum(jnp.where(valid[0], gbox[4][0], 0.0)) + \
            jnp.sum(jnp.where(valid[1], gbox[4][1], 0.0))

        bbox = jnp.zeros((L,), jnp.float32)
        for h in range(2):
            acc = jnp.zeros((L,), jnp.float32)
            for fi in range(4):
                tcomp = plsc.load_gather(tg, [fi * 32 + rank[h]])
                d = gbox[fi][h] - tcomp
                acc = acc + d * d
            bbox = bbox + jnp.where(valid[h], acc, 0.0)
        bb = jnp.sum(bbox)

        out_v = jnp.where(lane == 0, nval.astype(jnp.float32),
                          jnp.where(lane == 1, sx,
                                    jnp.where(lane == 2, bb, 0.0)))
        outrow[pl.ds(0, L)] = out_v
        pltpu.sync_copy(outrow, out_hbm.at[pl.ds(b * L, L)])


@jax.jit
def kernel(preds, targets):
    f32 = jnp.float32
    comp = jnp.transpose(preds, (2, 0, 1))
    padc = jnp.concatenate([
        jnp.full((2, B, NPAD - N), 2.0, f32),
        jnp.zeros((2, B, NPAD - N), f32),
        jnp.full((1, B, NPAD - N), -1e30, f32),
    ], axis=0)
    comp = jnp.concatenate([comp, padc], axis=2)
    comp_flat = comp.reshape(5 * B * NPAD)
    comp4 = comp.reshape(5, B, NROW, 128)
    tgt = jnp.transpose(targets, (0, 2, 1))
    tgt_flat = jnp.pad(tgt, ((0, 0), (0, 0), (0, 32 - M))).reshape(B * 4 * 32)
    tgt_h = tgt.reshape(B, 4, 2, MH).transpose(0, 2, 1, 3)
    tgt_tc = jnp.pad(tgt_h, ((0, 0), (0, 0), (0, 4), (0, 128 - MH)))
    tgt_tc = tgt_tc.reshape(B * 2, 8, 128)

    mx_out, idx_out = pl.pallas_call(
        _tc_scan_body,
        out_shape=(jax.ShapeDtypeStruct((B * 2, 8, 128), f32),
                   jax.ShapeDtypeStruct((B * 2, 8, 128), jnp.int32)),
        grid=(B * 2,),
        in_specs=[pl.BlockSpec((5, 1, NROW, 128), lambda i: (0, i // 2, 0, 0)),
                  pl.BlockSpec((1, 8, 128), lambda i: (i, 0, 0))],
        out_specs=(pl.BlockSpec((1, 8, 128), lambda i: (i, 0, 0)),
                   pl.BlockSpec((1, 8, 128), lambda i: (i, 0, 0))),
    )(comp4, tgt_tc)

    mesh = plsc.VectorSubcoreMesh(core_axis_name="c", subcore_axis_name="s")
    sc_call = pl.kernel(
        _sc_select_body,
        out_type=jax.ShapeDtypeStruct((B * L,), f32),
        mesh=mesh,
        compiler_params=pltpu.CompilerParams(needs_layout_passes=False),
        scratch_types=[
            pltpu.VMEM((NPAD,), f32),
            pltpu.VMEM((NPAD,), f32),
            pltpu.VMEM((NPAD,), f32),
            pltpu.VMEM((NPAD,), f32),
            pltpu.VMEM((NPAD,), f32),
            pltpu.VMEM((128,), f32),
            pltpu.VMEM((64,), f32),
            pltpu.VMEM((64,), jnp.int32),
            pltpu.VMEM((L,), f32),
        ],
    )
    sc_out = sc_call(comp_flat, tgt_flat,
                     mx_out.reshape(B * 2 * 1024),
                     idx_out.reshape(B * 2 * 1024)).reshape(B, L)

    s0 = mx_out.reshape(B, 2, 8, 128)[:, 0, 1, 0]
    n = sc_out[:, 0]
    sx = sc_out[:, 1]
    bb = sc_out[:, 2]
    conf_loss = (s0 - sx) / N
    bbox_loss = bb / (jnp.maximum(n, 1.0) * 4.0)
    per_batch = jnp.where(n > 0, bbox_loss + conf_loss, 0.0)
    return jnp.asarray(jnp.mean(per_batch), f32)


# speedup vs baseline: 1.2588x; 1.0395x over previous
import jax
import jax.numpy as jnp
from jax import lax
from jax.experimental import pallas as pl
from jax.experimental.pallas import tpu as pltpu
from jax.experimental.pallas import tpu_sc as plsc

B = 4
N = 20000
M = 20
MH = 5
NPROG = M // MH
NPAD = 20480
NROW = NPAD // 128
NK = NROW // 8
L = 16
BIG_IDX = 1 << 30


def _tc_scan_body(comp_ref, tgt_ref, mx_ref, idx_ref):
    f32 = jnp.float32
    j = pl.program_id(0) % NPROG
    mbase = j * MH
    sub = lax.broadcasted_iota(jnp.int32, (8, 128), 0)
    ln = lax.broadcasted_iota(jnp.int32, (8, 128), 1)
    sublin = sub * 128 + ln

    rmax = [jnp.full((8, 128), -1.0, f32) for _ in range(MH)]
    ridx = [jnp.zeros((8, 128), jnp.int32) for _ in range(MH)]
    for k in range(NK):
        px = comp_ref[0, 0, pl.ds(k * 8, 8), :]
        py = comp_ref[1, 0, pl.ds(k * 8, 8), :]
        pw = comp_ref[2, 0, pl.ds(k * 8, 8), :]
        ph = comp_ref[3, 0, pl.ds(k * 8, 8), :]
        px2 = px + pw
        py2 = py + ph
        pA = pw * ph
        kidx = k * 1024 + sublin
        for mi in range(MH):
            gx = tgt_ref[0, 0, mi]
            gy = tgt_ref[0, 1, mi]
            gw = tgt_ref[0, 2, mi]
            gh = tgt_ref[0, 3, mi]
            xa = jnp.maximum(px, gx)
            ya = jnp.maximum(py, gy)
            xb = jnp.minimum(px2, gx + gw)
            yb = jnp.minimum(py2, gy + gh)
            inter = jnp.maximum(xb - xa, 0.0) * jnp.maximum(yb - ya, 0.0)
            union = (pA + gw * gh) - inter
            upos = union > 0.0
            iou = jnp.where(upos, inter / jnp.where(upos, union, 1.0), 0.0)
            upd = iou > rmax[mi]
            rmax[mi] = jnp.where(upd, iou, rmax[mi])
            ridx[mi] = jnp.where(upd, kidx, ridx[mi])

    prev_m = jnp.where(j == 0, jnp.zeros((8, 128), f32), mx_ref[0])
    prev_i = jnp.where(j == 0, jnp.zeros((8, 128), jnp.int32), idx_ref[0])
    macc, iacc = prev_m, prev_i
    for mi in range(MH):
        mx = jnp.max(rmax[mi])
        mn = jnp.min(jnp.where(rmax[mi] == mx, ridx[mi], BIG_IDX))
        maskm = (sub == 0) & (ln == mbase + mi)
        macc = jnp.where(maskm, mx, macc)
        iacc = jnp.where(maskm, mn, iacc)
    mx_ref[0] = macc
    idx_ref[0] = iacc


def _tc_softplus_body(x_ref, o_ref):
    x = x_ref[0]
    g = jnp.maximum(x, 0.0) + jnp.log1p(jnp.exp(-jnp.abs(x)))
    o_ref[0] = jnp.full((8, 128), jnp.sum(g), jnp.float32)


def _sc_select_body(comp_hbm, tgt_hbm, mx_hbm, idx_hbm, out_hbm,
                    px, py, pw, ph, cf, tg, mrow, irow, outrow, sem):
    c = lax.axis_index("c")
    s = lax.axis_index("s")

    @pl.when((s == 0) | (s == 8))
    def _select():
        b = c * 2 + s // 8
        copies = [pltpu.async_copy(comp_hbm.at[k * B + b], ref, sem)
                  for k, ref in ((0, px), (1, py), (2, pw), (3, ph), (4, cf))]
        pltpu.sync_copy(tgt_hbm.at[pl.ds(b * 128, 128)], tg)
        pltpu.sync_copy(mx_hbm.at[pl.ds(b * 1024, 32)], mrow)
        pltpu.sync_copy(idx_hbm.at[pl.ds(b * 1024, 32)], irow)
        for cp in copies:
            cp.wait()

        lane = lax.broadcasted_iota(jnp.int32, (L,), 0)
        gmax = [mrow[pl.ds(0, L)], mrow[pl.ds(L, L)]]
        gidx = [irow[pl.ds(0, L)], irow[pl.ds(L, L)]]
        grow = [gidx[h] >> 7 for h in range(2)]
        gcol = [gidx[h] & 127 for h in range(2)]
        gbox = [[plsc.load_gather(ref, [grow[h], gcol[h]]) for h in range(2)]
                for ref in (px, py, pw, ph, cf)]

        hit = [gmax[h] > 0.5 for h in range(2)]
        hiti = [hit[h].astype(jnp.int32) for h in range(2)]

        mpos = [lane, lane + L]
        dup = [jnp.zeros((L,), jnp.bool_) for _ in range(2)]
        for mp in range(M):
            jm = gidx[mp // L][mp % L]
            hm = hiti[mp // L][mp % L] > 0
            for h in range(2):
                clash = hm & (gidx[h] == jm) & (mpos[h] > mp)
                dup[h] = dup[h] | clash
        valid = [hit[h] & (~dup[h]) for h in range(2)]
        key = [jnp.where(valid[h], gidx[h], BIG_IDX) for h in range(2)]

        rank = [jnp.zeros((L,), jnp.int32) for _ in range(2)]
        for mp in range(M):
            km = key[mp // L][mp % L]
            for h in range(2):
                rank[h] = rank[h] + (key[h] > km).astype(jnp.int32)

        nval = jnp.sum(valid[0].astype(jnp.int32)) + \
            jnp.sum(valid[1].astype(jnp.int32))
        sx = jnp.sum(jnp.where(valid[0], gbox[4][0], 0.0)) + \
            jnp.sum(jnp.where(valid[1], gbox[4][1], 0.0))

        bbox = jnp.zeros((L,), jnp.float32)
        for h in range(2):
            acc = jnp.zeros((L,), jnp.float32)
            for fi in range(4):
                tcomp = plsc.load_gather(tg, [fi * 32 + rank[h]])
                d = gbox[fi][h] - tcomp
                acc = acc + d * d
            bbox = bbox + jnp.where(valid[h], acc, 0.0)
        bb = jnp.sum(bbox)

        out_v = jnp.where(lane == 0, nval.astype(jnp.float32),
                          jnp.where(lane == 1, sx,
                                    jnp.where(lane == 2, bb, 0.0)))
        outrow[pl.ds(0, L)] = out_v
        pltpu.sync_copy(outrow, out_hbm.at[pl.ds(b * L, L)])


@jax.jit
def kernel(preds, targets):
    f32 = jnp.float32
    comp = jnp.transpose(preds, (2, 0, 1))
    padc = jnp.concatenate([
        jnp.full((2, B, NPAD - N), 2.0, f32),
        jnp.zeros((2, B, NPAD - N), f32),
        jnp.full((1, B, NPAD - N), -1e30, f32),
    ], axis=0)
    comp = jnp.concatenate([comp, padc], axis=2)
    comp4 = comp.reshape(5, B, NROW, 128)
    comp20 = comp4.reshape(5 * B, NROW, 128)
    tgt = jnp.transpose(targets, (0, 2, 1))
    tgt_flat = jnp.pad(tgt, ((0, 0), (0, 0), (0, 32 - M))).reshape(B * 4 * 32)
    tgt_h = tgt.reshape(B, 4, NPROG, MH).transpose(0, 2, 1, 3)
    tgt_tc = jnp.pad(tgt_h, ((0, 0), (0, 0), (0, 4), (0, 128 - MH)))
    tgt_tc = tgt_tc.reshape(B * NPROG, 8, 128)

    mx_out, idx_out = pl.pallas_call(
        _tc_scan_body,
        out_shape=(jax.ShapeDtypeStruct((B, 8, 128), f32),
                   jax.ShapeDtypeStruct((B, 8, 128), jnp.int32)),
        grid=(B * NPROG,),
        in_specs=[pl.BlockSpec((5, 1, NROW, 128),
                               lambda i: (0, i // NPROG, 0, 0)),
                  pl.BlockSpec((1, 8, 128), lambda i: (i, 0, 0))],
        out_specs=(pl.BlockSpec((1, 8, 128), lambda i: (i // NPROG, 0, 0)),
                   pl.BlockSpec((1, 8, 128), lambda i: (i // NPROG, 0, 0))),
    )(comp4, tgt_tc)

    s0 = pl.pallas_call(
        _tc_softplus_body,
        out_shape=jax.ShapeDtypeStruct((B, 8, 128), f32),
        grid=(B,),
        in_specs=[pl.BlockSpec((1, NROW, 128), lambda i: (i, 0, 0))],
        out_specs=pl.BlockSpec((1, 8, 128), lambda i: (i, 0, 0)),
    )(comp4[4])[:, 0, 0]

    mesh = plsc.VectorSubcoreMesh(core_axis_name="c", subcore_axis_name="s")
    sc_call = pl.kernel(
        _sc_select_body,
        out_type=jax.ShapeDtypeStruct((B * L,), f32),
        mesh=mesh,
        compiler_params=pltpu.CompilerParams(needs_layout_passes=False),
        scratch_types=[
            pltpu.VMEM((NROW, 128), f32),
            pltpu.VMEM((NROW, 128), f32),
            pltpu.VMEM((NROW, 128), f32),
            pltpu.VMEM((NROW, 128), f32),
            pltpu.VMEM((NROW, 128), f32),
            pltpu.VMEM((128,), f32),
            pltpu.VMEM((32,), f32),
            pltpu.VMEM((32,), jnp.int32),
            pltpu.VMEM((L,), f32),
            pltpu.SemaphoreType.DMA,
        ],
    )
    sc_out = sc_call(comp20, tgt_flat,
                     mx_out.reshape(B * 1024),
                     idx_out.reshape(B * 1024)).reshape(B, L)

    n = sc_out[:, 0]
    sx = sc_out[:, 1]
    bb = sc_out[:, 2]
    conf_loss = (s0 - sx) / N
    bbox_loss = bb / (jnp.maximum(n, 1.0) * 4.0)
    per_batch = jnp.where(n > 0, bbox_loss + conf_loss, 0.0)
    return jnp.asarray(jnp.mean(per_batch), f32)


# speedup vs baseline: 1.5481x; 1.2298x over previous
import jax
import jax.numpy as jnp
from jax import lax
from jax.experimental import pallas as pl
from jax.experimental.pallas import tpu as pltpu
from jax.experimental.pallas import tpu_sc as plsc

B = 4
N = 20000
M = 20
NPAD = 20480
NCHUNK = 8
CH = NPAD // NCHUNK
L = 16
NVEC = CH // L
GTG = 4
NSTREAM = 1
BIG_IDX = 1 << 30


def _sc_kernel_body(comp_hbm, tgt_hbm, out_hbm,
                    px, py, pw, ph, cf, px2, py2, pA,
                    tg, loc_f, loc_i, mrg_f, mrg_i, outrow,
                    shf, shi):
    c = lax.axis_index("c")
    s = lax.axis_index("s")
    b = c * 2 + s // NCHUNK
    chunk = s % NCHUNK
    base = chunk * CH

    for k, ref in ((0, px), (1, py), (2, pw), (3, ph), (4, cf)):
        off = (k * B + b) * NPAD + base
        pltpu.sync_copy(comp_hbm.at[pl.ds(off, CH)], ref)
    pltpu.sync_copy(tgt_hbm.at[pl.ds(b * 128, 128)], tg)

    def _pre(v, _):
        sl = pl.ds(v * L, L)
        px2[sl] = px[sl] + pw[sl]
        py2[sl] = py[sl] + ph[sl]
        pA[sl] = pw[sl] * ph[sl]
        return 0
    lax.fori_loop(0, NVEC, _pre, 0, unroll=4)

    lane = lax.broadcasted_iota(jnp.int32, (L,), 0)

    tgv = {}
    for ci in range(4):
        tgv[ci] = (tg[pl.ds(ci * 32, L)], tg[pl.ds(ci * 32 + L, L)])

    def _gt_scalar(ci, m):
        return tgv[ci][m // L][m % L]

    locm = [jnp.full((L,), -1.0, jnp.float32) for _ in range(2)]
    loci = [jnp.full((L,), base, jnp.int32) for _ in range(2)]

    for g in range(M // GTG):
        gts = []
        for mi in range(GTG):
            m = g * GTG + mi
            gx = _gt_scalar(0, m)
            gy = _gt_scalar(1, m)
            gw = _gt_scalar(2, m)
            gh = _gt_scalar(3, m)
            gts.append((gx, gy, gx + gw, gy + gh, gw * gh))

        half_v = NVEC // NSTREAM

        def _scan(v, carry):
            out = []
            for st in range(NSTREAM):
                vv = v + st * half_v
                idxv = base + vv * L + lane
                sl = pl.ds(vv * L, L)
                vx, vy, vx2, vy2, vA = px[sl], py[sl], px2[sl], py2[sl], pA[sl]
                for mi in range(GTG):
                    gx, gy, gx2, gy2, gA = gts[mi]
                    mcur, icur = carry[st * GTG + mi]
                    xa = jnp.maximum(vx, gx)
                    ya = jnp.maximum(vy, gy)
                    xb = jnp.minimum(vx2, gx2)
                    yb = jnp.minimum(vy2, gy2)
                    inter = jnp.maximum(xb - xa, 0.0) * \
                        jnp.maximum(yb - ya, 0.0)
                    iou = inter / ((vA + gA) - inter)
                    upd = iou > mcur
                    out.append((jnp.where(upd, iou, mcur),
                                jnp.where(upd, idxv, icur)))
            return tuple(out)

        init = tuple((jnp.full((L,), -1.0, jnp.float32),
                      jnp.full((L,), base, jnp.int32))
                     for _ in range(NSTREAM * GTG))
        res = lax.fori_loop(0, half_v, _scan, init, unroll=2)

        for mi in range(GTG):
            m = g * GTG + mi
            mvec, ivec = res[mi]
            for st in range(1, NSTREAM):
                m2, i2 = res[st * GTG + mi]
                tk0 = mvec >= m2
                mvec = jnp.where(tk0, mvec, m2)
                ivec = jnp.where(tk0, ivec, i2)
            mval = jnp.max(mvec)
            best = jnp.min(jnp.where(mvec == mval, ivec, BIG_IDX))
            locm[m // L] = jnp.where(lane == m % L, mval, locm[m // L])
            loci[m // L] = jnp.where(lane == m % L, best, loci[m // L])

    loc_f[pl.ds(0, L)] = locm[0]
    loc_f[pl.ds(L, L)] = locm[1]
    loc_i[pl.ds(0, L)] = loci[0]
    loc_i[pl.ds(L, L)] = loci[1]

    for half in range(2):
        rel = loci[half] - base
        for fi, ref in ((1, px), (2, py), (3, pw), (4, ph), (5, cf)):
            loc_f[pl.ds((fi * 2 + half) * L, L)] = plsc.load_gather(ref, [rel])

    pltpu.sync_copy(loc_f, shf.at[pl.ds(s * 384, 384)])
    pltpu.sync_copy(loc_i, shi.at[pl.ds(s * 32, 32)])
    plsc.subcore_barrier()

    @pl.when(s % NCHUNK == 0)
    def _merge():
        pltpu.sync_copy(shf.at[pl.ds(s * 384, NCHUNK * 384)], mrg_f)
        pltpu.sync_copy(shi.at[pl.ds(s * 32, NCHUNK * 32)], mrg_i)

        gmax = [jnp.full((L,), -1.0, jnp.float32) for _ in range(2)]
        gidx = [jnp.zeros((L,), jnp.int32) for _ in range(2)]
        gbox = [[jnp.zeros((L,), jnp.float32) for _ in range(2)]
                for _ in range(5)]
        for ci in range(NCHUNK):
            for half in range(2):
                cmax = mrg_f[pl.ds(ci * 384 + half * L, L)]
                cidx = mrg_i[pl.ds(ci * 32 + half * L, L)]
                upd = cmax > gmax[half]
                gmax[half] = jnp.where(upd, cmax, gmax[half])
                gidx[half] = jnp.where(upd, cidx, gidx[half])
                for fi in range(5):
                    cbox = mrg_f[pl.ds(ci * 384 + ((fi + 1) * 2 + half) * L, L)]
                    gbox[fi][half] = jnp.where(upd, cbox, gbox[fi][half])

        hit = [gmax[h] > 0.5 for h in range(2)]
        hiti = [hit[h].astype(jnp.int32) for h in range(2)]

        mpos = [lane, lane + L]
        dup = [jnp.zeros((L,), jnp.bool_) for _ in range(2)]
        for mp in range(M):
            jm = gidx[mp // L][mp % L]
            hm = hiti[mp // L][mp % L] > 0
            for h in range(2):
                clash = hm & (gidx[h] == jm) & (mpos[h] > mp)
                dup[h] = dup[h] | clash
        valid = [hit[h] & (~dup[h]) for h in range(2)]
        key = [jnp.where(valid[h], gidx[h], BIG_IDX) for h in range(2)]

        rank = [jnp.zeros((L,), jnp.int32) for _ in range(2)]
        for mp in range(M):
            km = key[mp // L][mp % L]
            for h in range(2):
                rank[h] = rank[h] + (key[h] > km).astype(jnp.int32)

        nval = jnp.sum(valid[0].astype(jnp.int32)) + \
            jnp.sum(valid[1].astype(jnp.int32))
        sx = jnp.sum(jnp.where(valid[0], gbox[4][0], 0.0)) + \
            jnp.sum(jnp.where(valid[1], gbox[4][1], 0.0))

        bbox = jnp.zeros((L,), jnp.float32)
        for h in range(2):
            acc = jnp.zeros((L,), jnp.float32)
            for fi in range(4):
                tcomp = plsc.load_gather(tg, [fi * 32 + rank[h]])
                d = gbox[fi][h] - tcomp
                acc = acc + d * d
            bbox = bbox + jnp.where(valid[h], acc, 0.0)
        bb = jnp.sum(bbox)

        out_v = jnp.where(lane == 0, nval.astype(jnp.float32),
                          jnp.where(lane == 1, sx,
                                    jnp.where(lane == 2, bb, 0.0)))
        outrow[pl.ds(0, L)] = out_v
        pltpu.sync_copy(outrow, out_hbm.at[pl.ds(b * L, L)])


def _tc_softplus_body(x_ref, o_ref):
    x = x_ref[0]
    g = jnp.maximum(x, 0.0) + jnp.log1p(jnp.exp(-jnp.abs(x)))
    o_ref[0] = jnp.full((8, 128), jnp.sum(g), jnp.float32)


@jax.jit
def kernel(preds, targets):
    f32 = jnp.float32
    comp = jnp.transpose(preds, (2, 0, 1))
    padc = jnp.concatenate([
        jnp.full((2, B, NPAD - N), 2.0, f32),
        jnp.zeros((2, B, NPAD - N), f32),
        jnp.full((1, B, NPAD - N), -1e30, f32),
    ], axis=0)
    comp = jnp.concatenate([comp, padc], axis=2)
    comp_flat = comp.reshape(5 * B * NPAD)
    tgt = jnp.transpose(targets, (0, 2, 1))
    tgt_flat = jnp.pad(tgt, ((0, 0), (0, 0), (0, 32 - M))).reshape(B * 4 * 32)

    mesh = plsc.VectorSubcoreMesh(core_axis_name="c", subcore_axis_name="s")
    sc_call = pl.kernel(
        _sc_kernel_body,
        out_type=jax.ShapeDtypeStruct((B * L,), f32),
        mesh=mesh,
        compiler_params=pltpu.CompilerParams(needs_layout_passes=False),
        scratch_types=[
            pltpu.VMEM((CH,), f32),
            pltpu.VMEM((CH,), f32),
            pltpu.VMEM((CH,), f32),
            pltpu.VMEM((CH,), f32),
            pltpu.VMEM((CH,), f32),
            pltpu.VMEM((CH,), f32),
            pltpu.VMEM((CH,), f32),
            pltpu.VMEM((CH,), f32),
            pltpu.VMEM((128,), f32),
            pltpu.VMEM((384,), f32),
            pltpu.VMEM((32,), jnp.int32),
            pltpu.VMEM((NCHUNK * 384,), f32),
            pltpu.VMEM((NCHUNK * 32,), jnp.int32),
            pltpu.VMEM((L,), f32),
            pltpu.VMEM_SHARED((16 * 384,), f32),
            pltpu.VMEM_SHARED((16 * 32,), jnp.int32),
        ],
    )
    sc_out = sc_call(comp_flat, tgt_flat).reshape(B, L)

    cf3 = comp[4].reshape(B, NPAD // 128, 128)
    s0_call = pl.pallas_call(
        _tc_softplus_body,
        out_shape=jax.ShapeDtypeStruct((B, 8, 128), f32),
        grid=(B,),
        in_specs=[pl.BlockSpec((1, NPAD // 128, 128), lambda i: (i, 0, 0))],
        out_specs=pl.BlockSpec((1, 8, 128), lambda i: (i, 0, 0)),
    )
    s0 = s0_call(cf3)[:, 0, 0]

    n = sc_out[:, 0]
    sx = sc_out[:, 1]
    bb = sc_out[:, 2]
    conf_loss = (s0 - sx) / N
    bbox_loss = bb / (jnp.maximum(n, 1.0) * 4.0)
    per_batch = jnp.where(n > 0, bbox_loss + conf_loss, 0.0)
    return jnp.asarray(jnp.mean(per_batch), f32)
